# baseline (speedup 1.0000x reference)
import functools

import jax
import jax.numpy as jnp
from jax import lax
from jax.experimental import pallas as pl
from jax.experimental.pallas import tpu as pltpu
from jax.experimental.pallas import tpu_sc as plsc

N = 10000
D = 128
HALF = 64
E = 320000

NC = 2
NT = 16
R_TILE = 160
ROWS = NT * R_TILE
EPAD = ROWS * 128
WCH = 16
ROWS_PER_TILE_N = N // NT

def _mesh():
    return plsc.VectorSubcoreMesh(core_axis_name="c", subcore_axis_name="s")


def _zero_fill(ref3, b):
    def body(i, _):
        ref3[b, i // 4, pl.ds((i % 4) * 16, 16)] = jnp.zeros((16,), jnp.float32)
        return 0
    lax.fori_loop(0, 512, body, 0)


def _deg_body(ei_ref, w_ref, out_ref, deg_s, idx_v, w_v, zb_v):
    c = lax.axis_index("c")
    s = lax.axis_index("s")

    def zbody(i, _):
        zb_v[pl.ds(i * 16, 16)] = jnp.zeros((16,), jnp.float32)
        return 0
    lax.fori_loop(0, 128, zbody, 0)
    @pl.when(s == 0)
    def _():
        for k in range(4):
            pltpu.sync_copy(zb_v, deg_s.at[pl.ds(k * 2048, 2048)])
        pltpu.sync_copy(zb_v.at[pl.ds(0, 1808)],
                        deg_s.at[pl.ds(4 * 2048, 1808)])

    plsc.subcore_barrier()

    base = (c * ROWS + s * R_TILE).astype(jnp.int32)
    pltpu.sync_copy(ei_ref.at[pl.ds(base, R_TILE)], idx_v)
    pltpu.sync_copy(w_ref.at[pl.ds(s * R_TILE, R_TILE)], w_v)

    def body(r, _):
        pltpu.sync_copy(w_v.at[r], deg_s.at[idx_v.at[r]], add=True)
        return 0
    lax.fori_loop(0, R_TILE, body, 0)

    plsc.subcore_barrier()

    off = s * 624
    @pl.when(s < 15)
    def _():
        pltpu.sync_copy(deg_s.at[pl.ds(off, 624)], zb_v.at[pl.ds(0, 624)])
        pltpu.sync_copy(zb_v.at[pl.ds(0, 624)],
                        out_ref.at[pl.ds(c * N + off, 624)])
    @pl.when(s == 15)
    def _():
        pltpu.sync_copy(deg_s.at[pl.ds(15 * 624, 640)], zb_v.at[pl.ds(0, 640)])
        pltpu.sync_copy(zb_v.at[pl.ds(0, 640)],
                        out_ref.at[pl.ds(c * N + 15 * 624, 640)])


_SC_PARAMS = pltpu.CompilerParams(use_tc_tiling_on_sc=False,
                                  needs_layout_passes=False)


def _degrees_sc(eiP, wP):
    return pl.kernel(
        _deg_body,
        out_type=jax.ShapeDtypeStruct((2 * N,), jnp.float32),
        mesh=_mesh(),
        compiler_params=_SC_PARAMS,
        scratch_types=[
            pltpu.VMEM_SHARED((N,), jnp.float32),
            pltpu.VMEM((R_TILE, 128), jnp.int32),
            pltpu.VMEM((R_TILE, 128), jnp.float32),
            pltpu.VMEM((2048,), jnp.float32),
        ],
    )(eiP, wP)


def _spmm_body(hs_ref, ei_ref, w_ref, agg_ref, agg_s,
               isrc_v, idst_v, wbuf, gbuf, sbuf, owidx_v, *sems):
    gsem = sems[0:2]
    ssem = sems[2:4]
    wsem = sems[4:6]
    c = lax.axis_index("c")
    s = lax.axis_index("s")

    nrow0 = s * ROWS_PER_TILE_N

    _zero_fill(sbuf, 0)
    for k in range(4):
        pltpu.sync_copy(sbuf.at[0], agg_s.at[pl.ds(nrow0 + k * 128, 128)])
    pltpu.sync_copy(sbuf.at[0, pl.ds(0, 113)],
                    agg_s.at[pl.ds(nrow0 + 512, 113)])

    ebase = (s * R_TILE).astype(jnp.int32)
    cv = jnp.full((16,), 0, jnp.int32) + c

    lane = lax.broadcasted_iota(jnp.int32, (16,), 0)
    for ch, q0 in enumerate((0, 128, 256, 384, 497)):
        for k in range(8):
            owidx_v[ch, pl.ds(k * 16, 16)] = (
                (nrow0 + q0 + k * 16 + lane) * 2 + cv)

    pltpu.sync_copy(ei_ref.at[pl.ds(ebase, R_TILE)], isrc_v)
    pltpu.sync_copy(ei_ref.at[pl.ds(ROWS + ebase, R_TILE)], idst_v)

    def adj(i, _):
        rr = i // 8
        kk = i % 8
        isrc_v[rr, pl.ds(kk * 16, 16)] = (
            isrc_v[rr, pl.ds(kk * 16, 16)] * 2 + cv)
        return 0
    lax.fori_loop(0, R_TILE * 8, adj, 0)

    plsc.subcore_barrier()

    for h in range(2):
        pltpu.async_copy(w_ref.at[pl.ds(ebase + h * WCH, WCH)], wbuf.at[h],
                         wsem[h])
    for b in range(2):
        pltpu.async_copy(hs_ref.at[isrc_v.at[b]], gbuf.at[b], gsem[b])

    evens = lax.broadcasted_iota(jnp.int32, (16,), 0) * 2
    odds = evens + 1
    himask = jnp.full((16,), -65536, jnp.int32)

    def pair_body(p, _):
        for h in range(2):
            chunk = p * 2 + h
            pltpu.make_async_copy(w_ref.at[pl.ds(ebase, WCH)], wbuf.at[h],
                                  wsem[h]).wait()

            def rr_body(q, _):
                for b in range(2):
                    r = chunk * WCH + q * 2 + b
                    rr = q * 2 + b
                    pltpu.make_async_copy(hs_ref.at[isrc_v.at[r]],
                                          gbuf.at[b], gsem[b]).wait()
                    @pl.when(r >= 2)
                    def _():
                        pltpu.make_async_copy(sbuf.at[b],
                                              agg_s.at[idst_v.at[r]],
                                              ssem[b]).wait()

                    def scale16(k, _):
                        wvec = wbuf[h, rr, pl.ds(k * 16, 16)]
                        for l in range(16):
                            wsp = wvec.at[jnp.full((16,), l, jnp.int32)].get(
                                mode="promise_in_bounds")
                            e = k * 16 + l
                            ev = jnp.full((16,), 0, jnp.int32) + e
                            for m in range(2):
                                raw = plsc.bitcast(
                                    gbuf[b, e, pl.ds(m * 32, 32)], jnp.int32)
                                lo = plsc.bitcast(
                                    lax.shift_left(raw, 16), jnp.float32)
                                hi = plsc.bitcast(
                                    lax.bitwise_and(raw, himask), jnp.float32)
                                bv = jnp.full((16,), 0, jnp.int32) + b
                                plsc.store_scatter(
                                    sbuf, [bv, ev, evens + m * 32], lo * wsp)
                                plsc.store_scatter(
                                    sbuf, [bv, ev, odds + m * 32], hi * wsp)
                        return 0
                    lax.fori_loop(0, 8, scale16, 0)

                    pltpu.async_copy(sbuf.at[b], agg_s.at[idst_v.at[r]],
                                     ssem[b], add=True)

                    nxt = r + 2
                    @pl.when(nxt < R_TILE)
                    def _():
                        pltpu.async_copy(hs_ref.at[isrc_v.at[nxt]],
                                         gbuf.at[b], gsem[b])
                return 0

            lax.fori_loop(0, WCH // 2, rr_body, 0)

            @pl.when(chunk + 2 < R_TILE // WCH)
            def _():
                pltpu.async_copy(
                    w_ref.at[pl.ds(ebase + (chunk + 2) * WCH, WCH)],
                    wbuf.at[h], wsem[h])
        return 0

    lax.fori_loop(0, R_TILE // (2 * WCH), pair_body, 0)

    for b in range(2):
        pltpu.make_async_copy(sbuf.at[b], agg_s.at[idst_v.at[0]],
                              ssem[b]).wait()

    plsc.subcore_barrier()

    for ch, q0 in enumerate((0, 128, 256, 384, 497)):
        pltpu.sync_copy(agg_s.at[pl.ds(nrow0 + q0, 128)], sbuf.at[ch % 2])
        pltpu.sync_copy(sbuf.at[ch % 2], agg_ref.at[owidx_v.at[ch]])


def _spmm_sc(hsF, eiP, wP):
    return pl.kernel(
        _spmm_body,
        out_type=jax.ShapeDtypeStruct((2 * N, HALF), jnp.float32),
        mesh=_mesh(),
        compiler_params=_SC_PARAMS,
        scratch_types=(
            [
                pltpu.VMEM_SHARED((N, HALF), jnp.float32),
                pltpu.VMEM((R_TILE, 128), jnp.int32),
                pltpu.VMEM((R_TILE, 128), jnp.int32),
                pltpu.VMEM((2, WCH, 128), jnp.float32),
                pltpu.VMEM((2, 128, HALF), jnp.bfloat16),
                pltpu.VMEM((2, 128, HALF), jnp.float32),
                pltpu.VMEM((5, 128), jnp.int32),
            ]
            + [pltpu.SemaphoreType.DMA] * 6
        ),
    )(hsF, eiP, wP)


def _prep_body(ds_ref, dd_ref, x_ref, xs_ref, a_ref, b_ref):
    a = jax.lax.rsqrt(jnp.maximum(ds_ref[...], 1e-6))
    b = jax.lax.rsqrt(jnp.maximum(dd_ref[...], 1e-6))
    a_ref[...] = a
    b_ref[...] = b
    xs_ref[...] = (x_ref[...] * a).astype(jnp.bfloat16)


def _prep(ds_col, dd_col, x):
    return pl.pallas_call(
        _prep_body,
        out_shape=[
            jax.ShapeDtypeStruct((N, D), jnp.bfloat16),
            jax.ShapeDtypeStruct((N, 1), jnp.float32),
            jax.ShapeDtypeStruct((N, 1), jnp.float32),
        ],
    )(ds_col, dd_col, x)


def _dense_body(agg_ref, a_ref, b_ref, w_ref, bias_ref, g_ref, be_ref,
                hprev_ref, h_ref, hs_ref):
    z = jnp.dot(agg_ref[...] * b_ref[...], w_ref[...],
                preferred_element_type=jnp.float32) + bias_ref[...]
    mu = jnp.mean(z, axis=0, keepdims=True)
    zc = z - mu
    var = jnp.mean(zc * zc, axis=0, keepdims=True)
    zn = zc * jax.lax.rsqrt(var + 1e-5) * g_ref[...] + be_ref[...]
    h = jnp.maximum(zn, 0.0) + hprev_ref[...]
    h_ref[...] = h
    hs_ref[...] = (h * a_ref[...]).astype(jnp.bfloat16)


def _dense(agg, a_col, b_col, w, bias, g, be, hprev):
    return pl.pallas_call(
        _dense_body,
        out_shape=[
            jax.ShapeDtypeStruct((N, D), jnp.float32),
            jax.ShapeDtypeStruct((N, D), jnp.bfloat16),
        ],
    )(agg, a_col, b_col, w, bias, g, be, hprev)


def _final_body(agg_ref, b_ref, w_ref, bias_ref, dw1_ref, db1_ref, dw2_ref,
                db2_ref, emb_ref, logits_ref):
    emb = jnp.dot(agg_ref[...] * b_ref[...], w_ref[...],
                  preferred_element_type=jnp.float32) + bias_ref[...]
    emb_ref[...] = emb
    t = jnp.maximum(jnp.dot(emb, dw1_ref[...],
                            preferred_element_type=jnp.float32)
                    + db1_ref[...], 0.0)
    logits_ref[...] = jnp.dot(t, dw2_ref[...],
                              preferred_element_type=jnp.float32) + db2_ref[...]


def _final(agg, b_col, w, bias, dw1, db1, dw2, db2):
    return pl.pallas_call(
        _final_body,
        out_shape=[
            jax.ShapeDtypeStruct((N, D), jnp.float32),
            jax.ShapeDtypeStruct((N, 2), jnp.float32),
        ],
    )(agg, b_col, w, bias, dw1, db1, dw2, db2)


def kernel(x, edge_index, edge_weight, W0, b0, W1, b1, W2, b2, g0, be0, g1,
           be1, dW1, db1, dW2, db2):
    src = edge_index[0]
    dst = edge_index[1]

    pad = EPAD - E
    padidx = (jnp.arange(pad, dtype=jnp.int32) * 16) % N
    srcP = jnp.concatenate([src, padidx])
    dstP = jnp.concatenate([dst, padidx])
    wpad = jnp.concatenate([edge_weight,
                            jnp.zeros((pad,), jnp.float32)])
    eiP = jnp.concatenate([srcP, dstP]).reshape(2 * ROWS, 128)
    wP = wpad.reshape(ROWS, 128)

    deg = _degrees_sc(eiP, wP)
    degT = deg.reshape(2, N).T
    ds_col = degT[:, 0:1]
    dd_col = degT[:, 1:2]
    xs, a_col, b_col = _prep(ds_col, dd_col, x)

    agg = _spmm_sc(xs.reshape(2 * N, HALF), eiP, wP).reshape(N, D)
    h0, h0s = _dense(agg, a_col, b_col, W0, b0.reshape(1, D),
                     g0.reshape(1, D), be0.reshape(1, D), x)
    agg = _spmm_sc(h0s.reshape(2 * N, HALF), eiP, wP).reshape(N, D)
    h1, h1s = _dense(agg, a_col, b_col, W1, b1.reshape(1, D),
                     g1.reshape(1, D), be1.reshape(1, D), h0)
    agg = _spmm_sc(h1s.reshape(2 * N, HALF), eiP, wP).reshape(N, D)
    emb, logits = _final(agg, b_col, W2, b2.reshape(1, D), dW1,
                         db1.reshape(1, 64), dW2, db2.reshape(1, 2))
    return (emb, logits)


# speedup vs baseline: 1.8472x; 1.8472x over previous
import functools

import jax
import jax.numpy as jnp
from jax import lax
from jax.experimental import pallas as pl
from jax.experimental.pallas import tpu as pltpu
from jax.experimental.pallas import tpu_sc as plsc

N = 10000
D = 128
HALF = 64
E = 320000

NC = 2
NT = 16
R_TILE = 160
ROWS = NT * R_TILE
EPAD = ROWS * 128
WCH = 16
ROWS_PER_TILE_N = N // NT

def _mesh():
    return plsc.VectorSubcoreMesh(core_axis_name="c", subcore_axis_name="s")


def _zero_fill(ref3, b):
    def body(i, _):
        ref3[b, i // 4, pl.ds((i % 4) * 16, 16)] = jnp.zeros((16,), jnp.float32)
        return 0
    lax.fori_loop(0, 512, body, 0)


def _deg_body(ei_ref, w_ref, out_ref, deg_s, idx_v, w_v, zb_v):
    c = lax.axis_index("c")
    s = lax.axis_index("s")

    def zbody(i, _):
        zb_v[pl.ds(i * 16, 16)] = jnp.zeros((16,), jnp.float32)
        return 0
    lax.fori_loop(0, 128, zbody, 0)
    @pl.when(s == 0)
    def _():
        for k in range(4):
            pltpu.sync_copy(zb_v, deg_s.at[pl.ds(k * 2048, 2048)])
        pltpu.sync_copy(zb_v.at[pl.ds(0, 1808)],
                        deg_s.at[pl.ds(4 * 2048, 1808)])

    plsc.subcore_barrier()

    base = (c * ROWS + s * R_TILE).astype(jnp.int32)
    pltpu.sync_copy(ei_ref.at[pl.ds(base, R_TILE)], idx_v)
    pltpu.sync_copy(w_ref.at[pl.ds(s * R_TILE, R_TILE)], w_v)

    def body(r, _):
        pltpu.sync_copy(w_v.at[r], deg_s.at[idx_v.at[r]], add=True)
        return 0
    lax.fori_loop(0, R_TILE, body, 0)

    plsc.subcore_barrier()

    off = s * 624
    @pl.when(s < 15)
    def _():
        pltpu.sync_copy(deg_s.at[pl.ds(off, 624)], zb_v.at[pl.ds(0, 624)])
        pltpu.sync_copy(zb_v.at[pl.ds(0, 624)],
                        out_ref.at[pl.ds(c * N + off, 624)])
    @pl.when(s == 15)
    def _():
        pltpu.sync_copy(deg_s.at[pl.ds(15 * 624, 640)], zb_v.at[pl.ds(0, 640)])
        pltpu.sync_copy(zb_v.at[pl.ds(0, 640)],
                        out_ref.at[pl.ds(c * N + 15 * 624, 640)])


_SC_PARAMS = pltpu.CompilerParams(use_tc_tiling_on_sc=False)


def _degrees_sc(eiP, wP):
    return pl.kernel(
        _deg_body,
        out_type=jax.ShapeDtypeStruct((2 * N,), jnp.float32),
        mesh=_mesh(),
        compiler_params=_SC_PARAMS,
        scratch_types=[
            pltpu.VMEM_SHARED((N,), jnp.float32),
            pltpu.VMEM((R_TILE, 128), jnp.int32),
            pltpu.VMEM((R_TILE, 128), jnp.float32),
            pltpu.VMEM((2048,), jnp.float32),
        ],
    )(eiP, wP)


def _spmm_body(hs_ref, ei_ref, w_ref, agg_ref, agg_s,
               isrc_v, idst_v, wbuf, gbuf, sbuf, owidx_v, *sems):
    gsem = sems[0:2]
    ssem = sems[2:4]
    wsem = sems[4:6]
    c = lax.axis_index("c")
    s = lax.axis_index("s")

    nrow0 = s * ROWS_PER_TILE_N

    _zero_fill(sbuf, 0)
    for k in range(4):
        pltpu.sync_copy(sbuf.at[0], agg_s.at[pl.ds(nrow0 + k * 128, 128)])
    pltpu.sync_copy(sbuf.at[0, pl.ds(0, 113)],
                    agg_s.at[pl.ds(nrow0 + 512, 113)])

    ebase = (s * R_TILE).astype(jnp.int32)
    cv = jnp.full((16,), 0, jnp.int32) + c

    lane = lax.broadcasted_iota(jnp.int32, (16,), 0)
    for ch, q0 in enumerate((0, 128, 256, 384, 497)):
        for k in range(8):
            owidx_v[ch, pl.ds(k * 16, 16)] = (
                (nrow0 + q0 + k * 16 + lane) * 2 + cv)

    pltpu.sync_copy(ei_ref.at[pl.ds(ebase, R_TILE)], isrc_v)
    pltpu.sync_copy(ei_ref.at[pl.ds(ROWS + ebase, R_TILE)], idst_v)

    def adj(i, _):
        rr = i // 8
        kk = i % 8
        isrc_v[rr, pl.ds(kk * 16, 16)] = (
            isrc_v[rr, pl.ds(kk * 16, 16)] * 2 + cv)
        return 0
    lax.fori_loop(0, R_TILE * 8, adj, 0)

    plsc.subcore_barrier()

    for h in range(2):
        pltpu.async_copy(w_ref.at[pl.ds(ebase + h * WCH, WCH)], wbuf.at[h],
                         wsem[h])
    for b in range(2):
        pltpu.async_copy(hs_ref.at[isrc_v.at[b]], gbuf.at[b], gsem[b])

    def pair_body(p, _):
        for h in range(2):
            chunk = p * 2 + h
            pltpu.make_async_copy(w_ref.at[pl.ds(ebase, WCH)], wbuf.at[h],
                                  wsem[h]).wait()

            def rr_body(q, _):
                for b in range(2):
                    r = chunk * WCH + q * 2 + b
                    rr = q * 2 + b
                    pltpu.make_async_copy(hs_ref.at[isrc_v.at[r]],
                                          gbuf.at[b], gsem[b]).wait()
                    @pl.when(r >= 2)
                    def _():
                        pltpu.make_async_copy(sbuf.at[b],
                                              agg_s.at[idst_v.at[r]],
                                              ssem[b]).wait()

                    def scale16(k, _):
                        wvec = wbuf[h, rr, pl.ds(k * 16, 16)]
                        for l in range(16):
                            wsp = wvec.at[jnp.full((16,), l, jnp.int32)].get(
                                mode="promise_in_bounds")
                            e = k * 16 + l
                            for m in range(4):
                                sbuf[b, e, pl.ds(m * 16, 16)] = (
                                    gbuf[b, e, pl.ds(m * 16, 16)] * wsp)
                        return 0
                    lax.fori_loop(0, 8, scale16, 0)

                    pltpu.async_copy(sbuf.at[b], agg_s.at[idst_v.at[r]],
                                     ssem[b], add=True)

                    nxt = r + 2
                    @pl.when(nxt < R_TILE)
                    def _():
                        pltpu.async_copy(hs_ref.at[isrc_v.at[nxt]],
                                         gbuf.at[b], gsem[b])
                return 0

            lax.fori_loop(0, WCH // 2, rr_body, 0)

            @pl.when(chunk + 2 < R_TILE // WCH)
            def _():
                pltpu.async_copy(
                    w_ref.at[pl.ds(ebase + (chunk + 2) * WCH, WCH)],
                    wbuf.at[h], wsem[h])
        return 0

    lax.fori_loop(0, R_TILE // (2 * WCH), pair_body, 0)

    for b in range(2):
        pltpu.make_async_copy(sbuf.at[b], agg_s.at[idst_v.at[0]],
                              ssem[b]).wait()

    plsc.subcore_barrier()

    for ch, q0 in enumerate((0, 128, 256, 384, 497)):
        pltpu.sync_copy(agg_s.at[pl.ds(nrow0 + q0, 128)], sbuf.at[ch % 2])
        pltpu.sync_copy(sbuf.at[ch % 2], agg_ref.at[owidx_v.at[ch]])


def _spmm_sc(hsF, eiP, wP):
    return pl.kernel(
        _spmm_body,
        out_type=jax.ShapeDtypeStruct((2 * N, HALF), jnp.float32),
        mesh=_mesh(),
        compiler_params=_SC_PARAMS,
        scratch_types=(
            [
                pltpu.VMEM_SHARED((N, HALF), jnp.float32),
                pltpu.VMEM((R_TILE, 128), jnp.int32),
                pltpu.VMEM((R_TILE, 128), jnp.int32),
                pltpu.VMEM((2, WCH, 128), jnp.float32),
                pltpu.VMEM((2, 128, HALF), jnp.float32),
                pltpu.VMEM((2, 128, HALF), jnp.float32),
                pltpu.VMEM((5, 128), jnp.int32),
            ]
            + [pltpu.SemaphoreType.DMA] * 6
        ),
    )(hsF, eiP, wP)


def _prep_body(ds_ref, dd_ref, x_ref, xs_ref, a_ref, b_ref):
    a = jax.lax.rsqrt(jnp.maximum(ds_ref[...], 1e-6))
    b = jax.lax.rsqrt(jnp.maximum(dd_ref[...], 1e-6))
    a_ref[...] = a
    b_ref[...] = b
    xs_ref[...] = x_ref[...] * a


def _prep(ds_col, dd_col, x):
    return pl.pallas_call(
        _prep_body,
        out_shape=[
            jax.ShapeDtypeStruct((N, D), jnp.float32),
            jax.ShapeDtypeStruct((N, 1), jnp.float32),
            jax.ShapeDtypeStruct((N, 1), jnp.float32),
        ],
    )(ds_col, dd_col, x)


def _dense_body(agg_ref, a_ref, b_ref, w_ref, bias_ref, g_ref, be_ref,
                hprev_ref, h_ref, hs_ref):
    z = jnp.dot(agg_ref[...] * b_ref[...], w_ref[...],
                preferred_element_type=jnp.float32) + bias_ref[...]
    mu = jnp.mean(z, axis=0, keepdims=True)
    zc = z - mu
    var = jnp.mean(zc * zc, axis=0, keepdims=True)
    zn = zc * jax.lax.rsqrt(var + 1e-5) * g_ref[...] + be_ref[...]
    h = jnp.maximum(zn, 0.0) + hprev_ref[...]
    h_ref[...] = h
    hs_ref[...] = h * a_ref[...]


def _dense(agg, a_col, b_col, w, bias, g, be, hprev):
    return pl.pallas_call(
        _dense_body,
        out_shape=[
            jax.ShapeDtypeStruct((N, D), jnp.float32),
            jax.ShapeDtypeStruct((N, D), jnp.float32),
        ],
    )(agg, a_col, b_col, w, bias, g, be, hprev)


def _final_body(agg_ref, b_ref, w_ref, bias_ref, dw1_ref, db1_ref, dw2_ref,
                db2_ref, emb_ref, logits_ref):
    emb = jnp.dot(agg_ref[...] * b_ref[...], w_ref[...],
                  preferred_element_type=jnp.float32) + bias_ref[...]
    emb_ref[...] = emb
    t = jnp.maximum(jnp.dot(emb, dw1_ref[...],
                            preferred_element_type=jnp.float32)
                    + db1_ref[...], 0.0)
    logits_ref[...] = jnp.dot(t, dw2_ref[...],
                              preferred_element_type=jnp.float32) + db2_ref[...]


def _final(agg, b_col, w, bias, dw1, db1, dw2, db2):
    return pl.pallas_call(
        _final_body,
        out_shape=[
            jax.ShapeDtypeStruct((N, D), jnp.float32),
            jax.ShapeDtypeStruct((N, 2), jnp.float32),
        ],
    )(agg, b_col, w, bias, dw1, db1, dw2, db2)


def kernel(x, edge_index, edge_weight, W0, b0, W1, b1, W2, b2, g0, be0, g1,
           be1, dW1, db1, dW2, db2):
    src = edge_index[0]
    dst = edge_index[1]

    pad = EPAD - E
    padidx = (jnp.arange(pad, dtype=jnp.int32) * 16) % N
    srcP = jnp.concatenate([src, padidx])
    dstP = jnp.concatenate([dst, padidx])
    wpad = jnp.concatenate([edge_weight,
                            jnp.zeros((pad,), jnp.float32)])
    eiP = jnp.concatenate([srcP, dstP]).reshape(2 * ROWS, 128)
    wP = wpad.reshape(ROWS, 128)

    deg = _degrees_sc(eiP, wP)
    degT = deg.reshape(2, N).T
    ds_col = degT[:, 0:1]
    dd_col = degT[:, 1:2]
    xs, a_col, b_col = _prep(ds_col, dd_col, x)

    agg = _spmm_sc(xs.reshape(2 * N, HALF), eiP, wP).reshape(N, D)
    h0, h0s = _dense(agg, a_col, b_col, W0, b0.reshape(1, D),
                     g0.reshape(1, D), be0.reshape(1, D), x)
    agg = _spmm_sc(h0s.reshape(2 * N, HALF), eiP, wP).reshape(N, D)
    h1, h1s = _dense(agg, a_col, b_col, W1, b1.reshape(1, D),
                     g1.reshape(1, D), be1.reshape(1, D), h0)
    agg = _spmm_sc(h1s.reshape(2 * N, HALF), eiP, wP).reshape(N, D)
    emb, logits = _final(agg, b_col, W2, b2.reshape(1, D), dW1,
                         db1.reshape(1, 64), dW2, db2.reshape(1, 2))
    return (emb, logits)
